# baseline (speedup 1.0000x reference)
import functools

import jax
import jax.numpy as jnp
from jax import lax
from jax.experimental import pallas as pl
from jax.experimental.pallas import tpu as pltpu
from jax.experimental.pallas import tpu_sc as plsc

N = 10000
E = 160000
DF = 128
DE = 16
H0, H1, H2 = 16, 64, 64

NC, NS = 2, 16
NW = NC * NS
C = 128
CPW = 40
EP = NW * CPW * C
NR = 10240
RPT = NR // NS
C5 = 64
CPW5 = EP // (NW * C5)


def _mm(a, b):
    return jnp.matmul(a, b, precision=lax.Precision.HIGHEST)


def _mmd(a, b):
    return jnp.matmul(a.astype(jnp.bfloat16), b.astype(jnp.bfloat16),
                      preferred_element_type=jnp.float32)


def _bf(a):
    return a.astype(jnp.bfloat16).astype(jnp.float32)

def _sc_mesh():
    return plsc.VectorSubcoreMesh(core_axis_name="c", subcore_axis_name="s")


def _gather_xs(x, srcp):
    @functools.partial(
        pl.kernel,
        out_type=jax.ShapeDtypeStruct((EP, DF), jnp.float32),
        mesh=_sc_mesh(),
        scratch_types=[
            pltpu.VMEM((C,), jnp.int32),
            pltpu.VMEM((C, DF), jnp.float32),
            pltpu.VMEM((C,), jnp.int32),
            pltpu.VMEM((C, DF), jnp.float32),
            pltpu.SemaphoreType.DMA,
            pltpu.SemaphoreType.DMA,
            pltpu.SemaphoreType.DMA,
            pltpu.SemaphoreType.DMA,
        ],
    )
    def k(x_hbm, src_hbm, out_hbm, idx_a, rows_a, idx_b, rows_b,
          gsem_a, gsem_b, wsem_a, wsem_b):
        wid = lax.axis_index("c") * NS + lax.axis_index("s")

        def issue_g(ci, idx_v, rows_v, gsem):
            base = (wid * CPW + ci) * C
            pltpu.sync_copy(src_hbm.at[pl.ds(base, C)], idx_v)
            pltpu.async_copy(x_hbm.at[idx_v], rows_v, gsem)

        def drain_g(rows_v, gsem):
            pltpu.make_async_copy(x_hbm.at[pl.ds(0, C)], rows_v, gsem).wait()

        def start_w(ci, rows_v, wsem):
            base = (wid * CPW + ci) * C
            pltpu.async_copy(rows_v, out_hbm.at[pl.ds(base, C)], wsem)

        def drain_w(rows_v, wsem):
            pltpu.make_async_copy(rows_v, out_hbm.at[pl.ds(0, C)], wsem).wait()

        issue_g(0, idx_a, rows_a, gsem_a)

        def pair(j, carry):
            @pl.when(j > 0)
            def _():
                drain_w(rows_b, wsem_b)

            issue_g(2 * j + 1, idx_b, rows_b, gsem_b)
            drain_g(rows_a, gsem_a)
            start_w(2 * j, rows_a, wsem_a)

            @pl.when(j < CPW // 2 - 1)
            def _():
                drain_w(rows_a, wsem_a)
                issue_g(2 * j + 2, idx_a, rows_a, gsem_a)

            drain_g(rows_b, gsem_b)
            start_w(2 * j + 1, rows_b, wsem_b)
            return carry

        lax.fori_loop(0, CPW // 2, pair, 0)
        drain_w(rows_a, wsem_a)
        drain_w(rows_b, wsem_b)

    return k(x, srcp)


def _edge_msg(eap, xs, W1, b1, W2, b2, W3q, b3q, S):
    BE = 512
    nblk = EP // BE

    def body(ea_ref, xs_ref, W1r, b1r, W2r, b2r, W3r, b3r, S_r, msg_ref):
        h = jnp.maximum(_mmd(ea_ref[...], W1r[...]) + b1r[...], 0.0)
        h = jnp.maximum(_mmd(h, W2r[...]) + b2r[...], 0.0)
        w = _mmd(h, W3r[...]) + b3r[...]
        xsb = _bf(xs_ref[...])
        wb = _bf(w)
        d = wb * jnp.concatenate([xsb] * H0, axis=1)
        msg16 = jnp.matmul(d, S_r[...], precision=lax.Precision.HIGHEST)
        msg_ref[...] = jnp.concatenate(
            [msg16, jnp.zeros((BE, DF - H0), jnp.float32)], axis=1)

    full = lambda a: pl.BlockSpec(a.shape, lambda i: (0,) * a.ndim)
    return pl.pallas_call(
        body,
        grid=(nblk,),
        in_specs=[
            pl.BlockSpec((BE, DE), lambda i: (i, 0)),
            pl.BlockSpec((BE, DF), lambda i: (i, 0)),
            full(W1), full(b1), full(W2), full(b2), full(W3q), full(b3q),
            full(S),
        ],
        out_specs=pl.BlockSpec((BE, DF), lambda i: (i, 0)),
        out_shape=jax.ShapeDtypeStruct((EP, DF), jnp.float32),
    )(eap, xs, W1, b1, W2, b2, W3q, b3q, S)


def _scatter_msg(msg, dstp, zeros128):
    @functools.partial(
        pl.kernel,
        out_type=jax.ShapeDtypeStruct((NC, NR, DF), jnp.float32),
        mesh=_sc_mesh(),
        scratch_types=[
            pltpu.VMEM((C,), jnp.int32),
            pltpu.VMEM((C, DF), jnp.float32),
            pltpu.VMEM((C,), jnp.int32),
            pltpu.VMEM((C, DF), jnp.float32),
            pltpu.SemaphoreType.DMA,
            pltpu.SemaphoreType.DMA,
            pltpu.VMEM_SHARED((NR, DF), jnp.float32),
        ],
    )
    def k(msg_hbm, dst_hbm, zero_hbm, out_hbm, idx_a, rows_a, idx_b, rows_b,
          gsem_a, gsem_b, acc):
        c = lax.axis_index("c")
        s = lax.axis_index("s")
        wid = c * NS + s
        pltpu.sync_copy(zero_hbm.at[pl.ds(s * RPT, RPT)],
                        acc.at[pl.ds(s * RPT, RPT)])
        plsc.subcore_barrier()

        def issue_g(ci, idx_v, rows_v, gsem):
            base = (wid * CPW + ci) * C
            pltpu.sync_copy(dst_hbm.at[pl.ds(base, C)], idx_v)
            pltpu.async_copy(msg_hbm.at[pl.ds(base, C)], rows_v, gsem)

        def drain_g(rows_v, gsem):
            pltpu.make_async_copy(msg_hbm.at[pl.ds(0, C)], rows_v, gsem).wait()

        issue_g(0, idx_a, rows_a, gsem_a)

        def pair(j, carry):
            issue_g(2 * j + 1, idx_b, rows_b, gsem_b)
            drain_g(rows_a, gsem_a)
            pltpu.sync_copy(rows_a, acc.at[idx_a], add=True)

            @pl.when(j < CPW // 2 - 1)
            def _():
                issue_g(2 * j + 2, idx_a, rows_a, gsem_a)

            drain_g(rows_b, gsem_b)
            pltpu.sync_copy(rows_b, acc.at[idx_b], add=True)
            return carry

        lax.fori_loop(0, CPW // 2, pair, 0)
        plsc.subcore_barrier()
        pltpu.sync_copy(acc.at[pl.ds(s * RPT, RPT)],
                        out_hbm.at[c, pl.ds(s * RPT, RPT)])

    return k(msg, dstp, zeros128)


def _gat_edges(xps, edt, srcp, dstp, zeros128):
    @functools.partial(
        pl.kernel,
        out_type=jax.ShapeDtypeStruct((NC, NR, DF), jnp.float32),
        mesh=_sc_mesh(),
        scratch_types=[
            pltpu.VMEM((C5,), jnp.int32),
            pltpu.VMEM((C5,), jnp.int32),
            pltpu.VMEM((C5, DF), jnp.float32),
            pltpu.VMEM((C5, DF), jnp.float32),
            pltpu.VMEM((C5,), jnp.int32),
            pltpu.VMEM((C5,), jnp.int32),
            pltpu.VMEM((C5, DF), jnp.float32),
            pltpu.VMEM((C5, DF), jnp.float32),
            pltpu.SemaphoreType.DMA,
            pltpu.SemaphoreType.DMA,
            pltpu.VMEM_SHARED((NR, DF), jnp.float32),
        ],
    )
    def k(xps_hbm, edt_hbm, src_hbm, dst_hbm, zero_hbm, out_hbm,
          sidx_a, didx_a, rows_a, ed_a, sidx_b, didx_b, rows_b, ed_b,
          sem_a, sem_b, acc):
        c = lax.axis_index("c")
        s = lax.axis_index("s")
        wid = c * NS + s
        pltpu.sync_copy(zero_hbm.at[pl.ds(s * RPT, RPT)],
                        acc.at[pl.ds(s * RPT, RPT)])
        plsc.subcore_barrier()

        def issue(ci, sidx_v, didx_v, rows_v, ed_v, sem):
            base = (wid * CPW5 + ci) * C5
            pltpu.sync_copy(src_hbm.at[pl.ds(base, C5)], sidx_v)
            pltpu.sync_copy(dst_hbm.at[pl.ds(base, C5)], didx_v)
            pltpu.async_copy(xps_hbm.at[sidx_v], rows_v, sem)
            pltpu.async_copy(edt_hbm.at[didx_v], ed_v, sem)

        def drain(rows_v, ed_v, sem):
            pltpu.make_async_copy(xps_hbm.at[pl.ds(0, C5)], rows_v, sem).wait()
            pltpu.make_async_copy(edt_hbm.at[pl.ds(0, C5)], ed_v, sem).wait()

        def process(rows_v, ed_v, didx_v):
            def edge(e, ec):
                t = rows_v[e, pl.ds(H1, 16)] + ed_v[e, pl.ds(0, 16)]
                al = jnp.exp(jnp.maximum(t, 0.2 * t))
                for c4 in range(H1 // 16):
                    sl = pl.ds(c4 * 16, 16)
                    rows_v[e, sl] = rows_v[e, sl] * al
                rows_v[e, pl.ds(H1, 16)] = al
                return ec

            lax.fori_loop(0, C5, edge, 0, unroll=4)
            pltpu.sync_copy(rows_v, acc.at[didx_v], add=True)

        issue(0, sidx_a, didx_a, rows_a, ed_a, sem_a)

        def pair(j, carry):
            issue(2 * j + 1, sidx_b, didx_b, rows_b, ed_b, sem_b)
            drain(rows_a, ed_a, sem_a)
            process(rows_a, ed_a, didx_a)

            @pl.when(j < CPW5 // 2 - 1)
            def _():
                issue(2 * j + 2, sidx_a, didx_a, rows_a, ed_a, sem_a)

            drain(rows_b, ed_b, sem_b)
            process(rows_b, ed_b, didx_b)
            return carry

        lax.fori_loop(0, CPW5 // 2, pair, 0)
        plsc.subcore_barrier()
        pltpu.sync_copy(acc.at[pl.ds(s * RPT, RPT)],
                        out_hbm.at[c, pl.ds(s * RPT, RPT)])

    return k(xps, edt, srcp, dstp, zeros128)


def _bn(t, g, b):
    mu = jnp.mean(t, axis=0, keepdims=True)
    var = jnp.mean((t - mu) * (t - mu), axis=0, keepdims=True)
    return (t - mu) / jnp.sqrt(var + 1e-5) * g + b


def _gat_tables(h, gW, gas, gad):
    xp = _mmd(h, gW)
    es = _mmd(xp, gas)
    ed = _mmd(xp, gad)
    xps = jnp.concatenate([xp, jnp.broadcast_to(es, (N, 16)),
                           jnp.zeros((N, 48), jnp.float32)], axis=1)
    edt = jnp.concatenate([jnp.broadcast_to(ed, (N, 16)),
                           jnp.zeros((N, 112), jnp.float32)], axis=1)
    return xps, edt


def _node1(x, accp, root, nbias, bg, bb, gW, gas, gad):
    def body(x_ref, acc_ref, root_r, nb_r, bg_r, bb_r, gW_r, gas_r, gad_r,
             xps_ref, edt_ref):
        agg = acc_ref[0] + acc_ref[1]
        t = _mmd(x_ref[...], root_r[...]) + agg + nb_r[...]
        h1 = jnp.maximum(_bn(t, bg_r[...], bb_r[...]), 0.0)
        xps, edt = _gat_tables(h1, gW_r[...], gas_r[...], gad_r[...])
        xps_ref[...] = xps
        edt_ref[...] = edt

    return pl.pallas_call(
        body,
        out_shape=(jax.ShapeDtypeStruct((N, DF), jnp.float32),
                   jax.ShapeDtypeStruct((N, DF), jnp.float32)),
    )(x, accp, root, nbias, bg, bb, gW, gas, gad)


def _gat_combine(acc_ref, xps_ref, edt_ref, gb):
    num = acc_ref[0, :, :H1] + acc_ref[1, :, :H1]
    den = acc_ref[0, :, H1:H1 + 1] + acc_ref[1, :, H1:H1 + 1]
    xp = xps_ref[:, :H1]
    t = xps_ref[:, H1:H1 + 1] + edt_ref[:, 0:1]
    al = jnp.exp(jnp.maximum(t, 0.2 * t))
    num = num + al * xp
    den = den + al
    return num / (den + 1e-16) + gb


def _node2(acc, xps1, edt1, g1b, bg, bb, gW, gas, gad):
    def body(acc_ref, xps_ref, edt_ref, g1b_r, bg_r, bb_r, gW_r, gas_r,
             gad_r, xps2_ref, edt2_ref):
        o = _gat_combine(acc_ref, xps_ref, edt_ref, g1b_r[...])
        h2 = jnp.maximum(_bn(o, bg_r[...], bb_r[...]), 0.0)
        xps, edt = _gat_tables(h2, gW_r[...], gas_r[...], gad_r[...])
        xps2_ref[...] = xps
        edt2_ref[...] = edt

    return pl.pallas_call(
        body,
        out_shape=(jax.ShapeDtypeStruct((N, DF), jnp.float32),
                   jax.ShapeDtypeStruct((N, DF), jnp.float32)),
    )(acc, xps1, edt1, g1b, bg, bb, gW, gas, gad)


def _head(acc, xps2, edt2, batch2d, g2b, bg, bb, fc1W, fc1b, bn1g, bn1b,
          fc2W, fc2b, bn2g, bn2b, fc3W, fc3b, ngraphs):
    def body(acc_ref, xps_ref, edt_ref, bat_ref, g2b_r, bg_r, bb_r,
             fc1W_r, fc1b_r, bn1g_r, bn1b_r, fc2W_r, fc2b_r, bn2g_r,
             bn2b_r, fc3W_r, fc3b_r, out_ref):
        o = _gat_combine(acc_ref, xps_ref, edt_ref, g2b_r[...])
        h3 = jnp.maximum(_bn(o, bg_r[...], bb_r[...]), 0.0)
        gid = lax.broadcasted_iota(jnp.int32, (ngraphs, N), 0)
        oh = (bat_ref[...] == gid).astype(jnp.float32)
        sums = _mm(oh, h3)
        cnt = jnp.sum(oh, axis=1, keepdims=True)
        g = sums / jnp.maximum(cnt, 1.0)
        g = jnp.maximum(
            _bn(_mmd(g, fc1W_r[...]) + fc1b_r[...], bn1g_r[...], bn1b_r[...]), 0.0)
        g = jnp.maximum(
            _bn(_mmd(g, fc2W_r[...]) + fc2b_r[...], bn2g_r[...], bn2b_r[...]), 0.0)
        out_ref[...] = _mmd(g, fc3W_r[...]) + fc3b_r[...]

    return pl.pallas_call(
        body,
        out_shape=jax.ShapeDtypeStruct((ngraphs, 64), jnp.float32),
    )(acc, xps2, edt2, batch2d, g2b, bg, bb, fc1W, fc1b, bn1g, bn1b,
      fc2W, fc2b, bn2g, bn2b, fc3W, fc3b)


def kernel(x, edge_index, edge_attr, batch, params):
    p = params
    src = edge_index[0]
    dst = edge_index[1]
    pad = EP - E
    srcp = jnp.concatenate([src, jnp.zeros((pad,), jnp.int32)])
    dstp = jnp.concatenate([dst, jnp.full((pad,), N, jnp.int32)])
    eap = jnp.concatenate([edge_attr, jnp.zeros((pad, DE), jnp.float32)])
    zeros128 = jnp.zeros((NR, DF), jnp.float32)
    batch2d = batch.reshape(1, N)

    r2 = lambda a: a.reshape(1, -1)
    W3q = p['emlp_W3'].reshape(64, DF, H0).transpose(0, 2, 1).reshape(64, DF * H0)
    b3q = p['emlp_b3'].reshape(DF, H0).T.reshape(1, DF * H0)

    S = (jnp.arange(DF * H0, dtype=jnp.int32)[:, None] // DF ==
         jnp.arange(H0, dtype=jnp.int32)[None, :]).astype(jnp.float32)
    xs = _gather_xs(x, srcp)
    msg = _edge_msg(eap, xs, p['emlp_W1'], r2(p['emlp_b1']),
                    p['emlp_W2'], r2(p['emlp_b2']), W3q, b3q, S)
    acc0 = _scatter_msg(msg, dstp, zeros128)
    xps1, edt1 = _node1(x, acc0[:, :N, :H0], p['nn_root'], r2(p['nn_bias']),
                        r2(p['bng1_g']), r2(p['bng1_b']),
                        p['gat1_W'], p['gat1_as'].reshape(-1, 1),
                        p['gat1_ad'].reshape(-1, 1))
    acc1 = _gat_edges(xps1, edt1, srcp, dstp, zeros128)
    xps2, edt2 = _node2(acc1[:, :N, :80], xps1, edt1[:, :8], r2(p['gat1_b']),
                        r2(p['bng2_g']), r2(p['bng2_b']),
                        p['gat2_W'], p['gat2_as'].reshape(-1, 1),
                        p['gat2_ad'].reshape(-1, 1))
    acc2 = _gat_edges(xps2, edt2, srcp, dstp, zeros128)
    out = _head(acc2[:, :N, :80], xps2, edt2[:, :8], batch2d, r2(p['gat2_b']),
                r2(p['bng3_g']), r2(p['bng3_b']),
                p['fc1_W'], r2(p['fc1_b']), r2(p['bn1_g']), r2(p['bn1_b']),
                p['fc2_W'], r2(p['fc2_b']), r2(p['bn2_g']), r2(p['bn2_b']),
                p['fc3_W'], r2(p['fc3_b']), 64)
    return out


# speedup vs baseline: 1.6357x; 1.6357x over previous
import functools

import jax
import jax.numpy as jnp
from jax import lax
from jax.experimental import pallas as pl
from jax.experimental.pallas import tpu as pltpu
from jax.experimental.pallas import tpu_sc as plsc

N = 10000
E = 160000
DF = 128
DE = 16
H0, H1, H2 = 16, 64, 64

NC, NS = 2, 16
NW = NC * NS
C = 128
CPW = 40
EP = NW * CPW * C
NR = 10240
RPT = NR // NS
C5 = 64
CPW5 = EP // (NW * C5)


def _mm(a, b):
    return jnp.matmul(a, b, precision=lax.Precision.HIGHEST)


def _mmd(a, b):
    return jnp.matmul(a.astype(jnp.bfloat16), b.astype(jnp.bfloat16),
                      preferred_element_type=jnp.float32)


def _bf(a):
    return a.astype(jnp.bfloat16).astype(jnp.float32)

def _sc_mesh():
    return plsc.VectorSubcoreMesh(core_axis_name="c", subcore_axis_name="s")


def _gather_xs(x, srcp):
    @functools.partial(
        pl.kernel,
        out_type=jax.ShapeDtypeStruct((EP, DF), jnp.float32),
        mesh=_sc_mesh(),
        scratch_types=[
            pltpu.VMEM((C,), jnp.int32),
            pltpu.VMEM((C, DF), jnp.float32),
            pltpu.VMEM((C,), jnp.int32),
            pltpu.VMEM((C, DF), jnp.float32),
            pltpu.SemaphoreType.DMA,
            pltpu.SemaphoreType.DMA,
            pltpu.SemaphoreType.DMA,
            pltpu.SemaphoreType.DMA,
        ],
    )
    def k(x_hbm, src_hbm, out_hbm, idx_a, rows_a, idx_b, rows_b,
          gsem_a, gsem_b, wsem_a, wsem_b):
        wid = lax.axis_index("c") * NS + lax.axis_index("s")

        def issue_g(ci, idx_v, rows_v, gsem):
            base = (wid * CPW + ci) * C
            pltpu.sync_copy(src_hbm.at[pl.ds(base, C)], idx_v)
            pltpu.async_copy(x_hbm.at[idx_v], rows_v, gsem)

        def drain_g(rows_v, gsem):
            pltpu.make_async_copy(x_hbm.at[pl.ds(0, C)], rows_v, gsem).wait()

        def start_w(ci, rows_v, wsem):
            base = (wid * CPW + ci) * C
            pltpu.async_copy(rows_v, out_hbm.at[pl.ds(base, C)], wsem)

        def drain_w(rows_v, wsem):
            pltpu.make_async_copy(rows_v, out_hbm.at[pl.ds(0, C)], wsem).wait()

        issue_g(0, idx_a, rows_a, gsem_a)

        def pair(j, carry):
            @pl.when(j > 0)
            def _():
                drain_w(rows_b, wsem_b)

            issue_g(2 * j + 1, idx_b, rows_b, gsem_b)
            drain_g(rows_a, gsem_a)
            start_w(2 * j, rows_a, wsem_a)

            @pl.when(j < CPW // 2 - 1)
            def _():
                drain_w(rows_a, wsem_a)
                issue_g(2 * j + 2, idx_a, rows_a, gsem_a)

            drain_g(rows_b, gsem_b)
            start_w(2 * j + 1, rows_b, wsem_b)
            return carry

        lax.fori_loop(0, CPW // 2, pair, 0)
        drain_w(rows_a, wsem_a)
        drain_w(rows_b, wsem_b)

    return k(x, srcp)


def _edge_msg(eap, xs, W1, b1, W2, b2, W3q, b3q, S):
    BE = 512
    nblk = EP // BE

    def body(ea_ref, xs_ref, W1r, b1r, W2r, b2r, W3r, b3r, S_r, msg_ref):
        h = jnp.maximum(_mmd(ea_ref[...], W1r[...]) + b1r[...], 0.0)
        h = jnp.maximum(_mmd(h, W2r[...]) + b2r[...], 0.0)
        w = _mmd(h, W3r[...]) + b3r[...]
        xsb = _bf(xs_ref[...])
        wb = _bf(w)
        cols = [
            jnp.sum(wb[:, o * DF:(o + 1) * DF] * xsb, axis=1, keepdims=True)
            for o in range(H0)
        ]
        cols.append(jnp.zeros((BE, DF - H0), jnp.float32))
        msg_ref[...] = jnp.concatenate(cols, axis=1)

    full = lambda a: pl.BlockSpec(a.shape, lambda i: (0,) * a.ndim)
    return pl.pallas_call(
        body,
        grid=(nblk,),
        in_specs=[
            pl.BlockSpec((BE, DE), lambda i: (i, 0)),
            pl.BlockSpec((BE, DF), lambda i: (i, 0)),
            full(W1), full(b1), full(W2), full(b2), full(W3q), full(b3q),
            full(S),
        ],
        out_specs=pl.BlockSpec((BE, DF), lambda i: (i, 0)),
        out_shape=jax.ShapeDtypeStruct((EP, DF), jnp.float32),
    )(eap, xs, W1, b1, W2, b2, W3q, b3q, S)


def _scatter_msg(msg, dstp, zeros128):
    @functools.partial(
        pl.kernel,
        out_type=jax.ShapeDtypeStruct((NC, NR, DF), jnp.float32),
        mesh=_sc_mesh(),
        scratch_types=[
            pltpu.VMEM((C,), jnp.int32),
            pltpu.VMEM((C, DF), jnp.float32),
            pltpu.VMEM((C,), jnp.int32),
            pltpu.VMEM((C, DF), jnp.float32),
            pltpu.SemaphoreType.DMA,
            pltpu.SemaphoreType.DMA,
            pltpu.VMEM_SHARED((NR, DF), jnp.float32),
        ],
    )
    def k(msg_hbm, dst_hbm, zero_hbm, out_hbm, idx_a, rows_a, idx_b, rows_b,
          gsem_a, gsem_b, acc):
        c = lax.axis_index("c")
        s = lax.axis_index("s")
        wid = c * NS + s
        pltpu.sync_copy(zero_hbm.at[pl.ds(s * RPT, RPT)],
                        acc.at[pl.ds(s * RPT, RPT)])
        plsc.subcore_barrier()

        def issue_g(ci, idx_v, rows_v, gsem):
            base = (wid * CPW + ci) * C
            pltpu.sync_copy(dst_hbm.at[pl.ds(base, C)], idx_v)
            pltpu.async_copy(msg_hbm.at[pl.ds(base, C)], rows_v, gsem)

        def drain_g(rows_v, gsem):
            pltpu.make_async_copy(msg_hbm.at[pl.ds(0, C)], rows_v, gsem).wait()

        issue_g(0, idx_a, rows_a, gsem_a)

        def pair(j, carry):
            issue_g(2 * j + 1, idx_b, rows_b, gsem_b)
            drain_g(rows_a, gsem_a)
            pltpu.sync_copy(rows_a, acc.at[idx_a], add=True)

            @pl.when(j < CPW // 2 - 1)
            def _():
                issue_g(2 * j + 2, idx_a, rows_a, gsem_a)

            drain_g(rows_b, gsem_b)
            pltpu.sync_copy(rows_b, acc.at[idx_b], add=True)
            return carry

        lax.fori_loop(0, CPW // 2, pair, 0)
        plsc.subcore_barrier()
        pltpu.sync_copy(acc.at[pl.ds(s * RPT, RPT)],
                        out_hbm.at[c, pl.ds(s * RPT, RPT)])

    return k(msg, dstp, zeros128)


def _gat_edges(xps, edt, srcp, dstp, zeros128):
    @functools.partial(
        pl.kernel,
        out_type=jax.ShapeDtypeStruct((NC, NR, DF), jnp.float32),
        mesh=_sc_mesh(),
        scratch_types=[
            pltpu.VMEM((C5,), jnp.int32),
            pltpu.VMEM((C5,), jnp.int32),
            pltpu.VMEM((C5, DF), jnp.float32),
            pltpu.VMEM((C5, DF), jnp.float32),
            pltpu.VMEM((C5,), jnp.int32),
            pltpu.VMEM((C5,), jnp.int32),
            pltpu.VMEM((C5, DF), jnp.float32),
            pltpu.VMEM((C5, DF), jnp.float32),
            pltpu.SemaphoreType.DMA,
            pltpu.SemaphoreType.DMA,
            pltpu.VMEM_SHARED((NR, DF), jnp.float32),
        ],
    )
    def k(xps_hbm, edt_hbm, src_hbm, dst_hbm, zero_hbm, out_hbm,
          sidx_a, didx_a, rows_a, ed_a, sidx_b, didx_b, rows_b, ed_b,
          sem_a, sem_b, acc):
        c = lax.axis_index("c")
        s = lax.axis_index("s")
        wid = c * NS + s
        pltpu.sync_copy(zero_hbm.at[pl.ds(s * RPT, RPT)],
                        acc.at[pl.ds(s * RPT, RPT)])
        plsc.subcore_barrier()

        def issue(ci, sidx_v, didx_v, rows_v, ed_v, sem):
            base = (wid * CPW5 + ci) * C5
            pltpu.sync_copy(src_hbm.at[pl.ds(base, C5)], sidx_v)
            pltpu.sync_copy(dst_hbm.at[pl.ds(base, C5)], didx_v)
            pltpu.async_copy(xps_hbm.at[sidx_v], rows_v, sem)
            pltpu.async_copy(edt_hbm.at[didx_v], ed_v, sem)

        def drain(rows_v, ed_v, sem):
            pltpu.make_async_copy(xps_hbm.at[pl.ds(0, C5)], rows_v, sem).wait()
            pltpu.make_async_copy(edt_hbm.at[pl.ds(0, C5)], ed_v, sem).wait()

        def process(rows_v, ed_v, didx_v):
            def edge(e, ec):
                t = rows_v[e, pl.ds(H1, 16)] + ed_v[e, pl.ds(0, 16)]
                al = jnp.exp(jnp.maximum(t, 0.2 * t))
                for c4 in range(H1 // 16):
                    sl = pl.ds(c4 * 16, 16)
                    rows_v[e, sl] = rows_v[e, sl] * al
                rows_v[e, pl.ds(H1, 16)] = al
                return ec

            lax.fori_loop(0, C5, edge, 0, unroll=4)
            pltpu.sync_copy(rows_v, acc.at[didx_v], add=True)

        issue(0, sidx_a, didx_a, rows_a, ed_a, sem_a)

        def pair(j, carry):
            issue(2 * j + 1, sidx_b, didx_b, rows_b, ed_b, sem_b)
            drain(rows_a, ed_a, sem_a)
            process(rows_a, ed_a, didx_a)

            @pl.when(j < CPW5 // 2 - 1)
            def _():
                issue(2 * j + 2, sidx_a, didx_a, rows_a, ed_a, sem_a)

            drain(rows_b, ed_b, sem_b)
            process(rows_b, ed_b, didx_b)
            return carry

        lax.fori_loop(0, CPW5 // 2, pair, 0)
        plsc.subcore_barrier()
        pltpu.sync_copy(acc.at[pl.ds(s * RPT, RPT)],
                        out_hbm.at[c, pl.ds(s * RPT, RPT)])

    return k(xps, edt, srcp, dstp, zeros128)


def _bn(t, g, b):
    mu = jnp.mean(t, axis=0, keepdims=True)
    var = jnp.mean((t - mu) * (t - mu), axis=0, keepdims=True)
    return (t - mu) / jnp.sqrt(var + 1e-5) * g + b


def _gat_tables(h, gW, gas, gad):
    xp = _mmd(h, gW)
    es = _mmd(xp, gas)
    ed = _mmd(xp, gad)
    xps = jnp.concatenate([xp, jnp.broadcast_to(es, (N, 16)),
                           jnp.zeros((N, 48), jnp.float32)], axis=1)
    edt = jnp.concatenate([jnp.broadcast_to(ed, (N, 16)),
                           jnp.zeros((N, 112), jnp.float32)], axis=1)
    return xps, edt


def _node1(x, accp, root, nbias, bg, bb, gW, gas, gad):
    def body(x_ref, acc_ref, root_r, nb_r, bg_r, bb_r, gW_r, gas_r, gad_r,
             xps_ref, edt_ref):
        agg = acc_ref[0] + acc_ref[1]
        t = _mmd(x_ref[...], root_r[...]) + agg + nb_r[...]
        h1 = jnp.maximum(_bn(t, bg_r[...], bb_r[...]), 0.0)
        xps, edt = _gat_tables(h1, gW_r[...], gas_r[...], gad_r[...])
        xps_ref[...] = xps
        edt_ref[...] = edt

    return pl.pallas_call(
        body,
        out_shape=(jax.ShapeDtypeStruct((N, DF), jnp.float32),
                   jax.ShapeDtypeStruct((N, DF), jnp.float32)),
    )(x, accp, root, nbias, bg, bb, gW, gas, gad)


def _gat_combine(acc_ref, xps_ref, edt_ref, gb):
    num = acc_ref[0, :, :H1] + acc_ref[1, :, :H1]
    den = acc_ref[0, :, H1:H1 + 1] + acc_ref[1, :, H1:H1 + 1]
    xp = xps_ref[:, :H1]
    t = xps_ref[:, H1:H1 + 1] + edt_ref[:, 0:1]
    al = jnp.exp(jnp.maximum(t, 0.2 * t))
    num = num + al * xp
    den = den + al
    return num / (den + 1e-16) + gb


def _node2(acc, xps1, edt1, g1b, bg, bb, gW, gas, gad):
    def body(acc_ref, xps_ref, edt_ref, g1b_r, bg_r, bb_r, gW_r, gas_r,
             gad_r, xps2_ref, edt2_ref):
        o = _gat_combine(acc_ref, xps_ref, edt_ref, g1b_r[...])
        h2 = jnp.maximum(_bn(o, bg_r[...], bb_r[...]), 0.0)
        xps, edt = _gat_tables(h2, gW_r[...], gas_r[...], gad_r[...])
        xps2_ref[...] = xps
        edt2_ref[...] = edt

    return pl.pallas_call(
        body,
        out_shape=(jax.ShapeDtypeStruct((N, DF), jnp.float32),
                   jax.ShapeDtypeStruct((N, DF), jnp.float32)),
    )(acc, xps1, edt1, g1b, bg, bb, gW, gas, gad)


def _head(acc, xps2, edt2, batch2d, g2b, bg, bb, fc1W, fc1b, bn1g, bn1b,
          fc2W, fc2b, bn2g, bn2b, fc3W, fc3b, ngraphs):
    def body(acc_ref, xps_ref, edt_ref, bat_ref, g2b_r, bg_r, bb_r,
             fc1W_r, fc1b_r, bn1g_r, bn1b_r, fc2W_r, fc2b_r, bn2g_r,
             bn2b_r, fc3W_r, fc3b_r, out_ref):
        o = _gat_combine(acc_ref, xps_ref, edt_ref, g2b_r[...])
        h3 = jnp.maximum(_bn(o, bg_r[...], bb_r[...]), 0.0)
        gid = lax.broadcasted_iota(jnp.int32, (ngraphs, N), 0)
        oh = (bat_ref[...] == gid).astype(jnp.float32)
        sums = _mm(oh, h3)
        cnt = jnp.sum(oh, axis=1, keepdims=True)
        g = sums / jnp.maximum(cnt, 1.0)
        g = jnp.maximum(
            _bn(_mmd(g, fc1W_r[...]) + fc1b_r[...], bn1g_r[...], bn1b_r[...]), 0.0)
        g = jnp.maximum(
            _bn(_mmd(g, fc2W_r[...]) + fc2b_r[...], bn2g_r[...], bn2b_r[...]), 0.0)
        out_ref[...] = _mmd(g, fc3W_r[...]) + fc3b_r[...]

    return pl.pallas_call(
        body,
        out_shape=jax.ShapeDtypeStruct((ngraphs, 64), jnp.float32),
    )(acc, xps2, edt2, batch2d, g2b, bg, bb, fc1W, fc1b, bn1g, bn1b,
      fc2W, fc2b, bn2g, bn2b, fc3W, fc3b)


def kernel(x, edge_index, edge_attr, batch, params):
    p = params
    src = edge_index[0]
    dst = edge_index[1]
    pad = EP - E
    srcp = jnp.concatenate([src, jnp.zeros((pad,), jnp.int32)])
    dstp = jnp.concatenate([dst, jnp.full((pad,), N, jnp.int32)])
    eap = jnp.concatenate([edge_attr, jnp.zeros((pad, DE), jnp.float32)])
    zeros128 = jnp.zeros((NR, DF), jnp.float32)
    batch2d = batch.reshape(1, N)

    r2 = lambda a: a.reshape(1, -1)
    W3q = p['emlp_W3'].reshape(64, DF, H0).transpose(0, 2, 1).reshape(64, DF * H0)
    b3q = p['emlp_b3'].reshape(DF, H0).T.reshape(1, DF * H0)

    S = (jnp.arange(DF * H0, dtype=jnp.int32)[:, None] // DF ==
         jnp.arange(H0, dtype=jnp.int32)[None, :]).astype(jnp.float32)
    xs = _gather_xs(x, srcp)
    msg = _edge_msg(eap, xs, p['emlp_W1'], r2(p['emlp_b1']),
                    p['emlp_W2'], r2(p['emlp_b2']), W3q, b3q, S)
    acc0 = _scatter_msg(msg, dstp, zeros128)
    xps1, edt1 = _node1(x, acc0[:, :N, :H0], p['nn_root'], r2(p['nn_bias']),
                        r2(p['bng1_g']), r2(p['bng1_b']),
                        p['gat1_W'], p['gat1_as'].reshape(-1, 1),
                        p['gat1_ad'].reshape(-1, 1))
    acc1 = _gat_edges(xps1, edt1, srcp, dstp, zeros128)
    xps2, edt2 = _node2(acc1[:, :N, :80], xps1, edt1[:, :8], r2(p['gat1_b']),
                        r2(p['bng2_g']), r2(p['bng2_b']),
                        p['gat2_W'], p['gat2_as'].reshape(-1, 1),
                        p['gat2_ad'].reshape(-1, 1))
    acc2 = _gat_edges(xps2, edt2, srcp, dstp, zeros128)
    out = _head(acc2[:, :N, :80], xps2, edt2[:, :8], batch2d, r2(p['gat2_b']),
                r2(p['bng3_g']), r2(p['bng3_b']),
                p['fc1_W'], r2(p['fc1_b']), r2(p['bn1_g']), r2(p['bn1_b']),
                p['fc2_W'], r2(p['fc2_b']), r2(p['bn2_g']), r2(p['bn2_b']),
                p['fc3_W'], r2(p['fc3_b']), 64)
    return out


# speedup vs baseline: 1.6876x; 1.0317x over previous
import functools

import jax
import jax.numpy as jnp
from jax import lax
from jax.experimental import pallas as pl
from jax.experimental.pallas import tpu as pltpu
from jax.experimental.pallas import tpu_sc as plsc

N = 10000
E = 160000
DF = 128
DE = 16
H0, H1, H2 = 16, 64, 64

NC, NS = 2, 16
NW = NC * NS
C = 128
CPW = 40
EP = NW * CPW * C
NR = 10240
RPT = NR // NS
C5 = 64
CPW5 = EP // (NW * C5)


def _mm(a, b):
    return jnp.matmul(a, b, precision=lax.Precision.HIGHEST)


def _mmd(a, b):
    return jnp.matmul(a.astype(jnp.bfloat16), b.astype(jnp.bfloat16),
                      preferred_element_type=jnp.float32)


def _bf(a):
    return a.astype(jnp.bfloat16).astype(jnp.float32)

def _sc_mesh():
    return plsc.VectorSubcoreMesh(core_axis_name="c", subcore_axis_name="s")


def _gather_xs(x, srcp):
    @functools.partial(
        pl.kernel,
        out_type=jax.ShapeDtypeStruct((EP, DF), jnp.float32),
        mesh=_sc_mesh(),
        scratch_types=[
            pltpu.VMEM((C,), jnp.int32),
            pltpu.VMEM((C, DF), jnp.float32),
            pltpu.VMEM((C,), jnp.int32),
            pltpu.VMEM((C, DF), jnp.float32),
            pltpu.SemaphoreType.DMA,
            pltpu.SemaphoreType.DMA,
            pltpu.SemaphoreType.DMA,
            pltpu.SemaphoreType.DMA,
        ],
    )
    def k(x_hbm, src_hbm, out_hbm, idx_a, rows_a, idx_b, rows_b,
          gsem_a, gsem_b, wsem_a, wsem_b):
        wid = lax.axis_index("c") * NS + lax.axis_index("s")

        def issue_g(ci, idx_v, rows_v, gsem):
            base = (wid * CPW + ci) * C
            pltpu.sync_copy(src_hbm.at[pl.ds(base, C)], idx_v)
            pltpu.async_copy(x_hbm.at[idx_v], rows_v, gsem)

        def drain_g(rows_v, gsem):
            pltpu.make_async_copy(x_hbm.at[pl.ds(0, C)], rows_v, gsem).wait()

        def start_w(ci, rows_v, wsem):
            base = (wid * CPW + ci) * C
            pltpu.async_copy(rows_v, out_hbm.at[pl.ds(base, C)], wsem)

        def drain_w(rows_v, wsem):
            pltpu.make_async_copy(rows_v, out_hbm.at[pl.ds(0, C)], wsem).wait()

        issue_g(0, idx_a, rows_a, gsem_a)

        def pair(j, carry):
            @pl.when(j > 0)
            def _():
                drain_w(rows_b, wsem_b)

            issue_g(2 * j + 1, idx_b, rows_b, gsem_b)
            drain_g(rows_a, gsem_a)
            start_w(2 * j, rows_a, wsem_a)

            @pl.when(j < CPW // 2 - 1)
            def _():
                drain_w(rows_a, wsem_a)
                issue_g(2 * j + 2, idx_a, rows_a, gsem_a)

            drain_g(rows_b, gsem_b)
            start_w(2 * j + 1, rows_b, wsem_b)
            return carry

        lax.fori_loop(0, CPW // 2, pair, 0)
        drain_w(rows_a, wsem_a)
        drain_w(rows_b, wsem_b)

    return k(x, srcp)


def _edge_msg(eap, xs, W1, b1, W2, b2, W3q, b3q, S):
    BE = 512
    nblk = EP // BE

    def body(ea_ref, xs_ref, W1r, b1r, W2r, b2r, W3r, b3r, S_r, msg_ref):
        h = jnp.maximum(_mmd(ea_ref[...], W1r[...]) + b1r[...], 0.0)
        h = jnp.maximum(_mmd(h, W2r[...]) + b2r[...], 0.0)
        w = _mmd(h, W3r[...]) + b3r[...]
        xsb = _bf(xs_ref[...])
        wb = _bf(w)
        cols = [
            jnp.sum(wb[:, o * DF:(o + 1) * DF] * xsb, axis=1, keepdims=True)
            for o in range(H0)
        ]
        cols.append(jnp.zeros((BE, DF - H0), jnp.float32))
        msg_ref[...] = jnp.concatenate(cols, axis=1)

    full = lambda a: pl.BlockSpec(a.shape, lambda i: (0,) * a.ndim)
    return pl.pallas_call(
        body,
        grid=(nblk,),
        in_specs=[
            pl.BlockSpec((BE, DE), lambda i: (i, 0)),
            pl.BlockSpec((BE, DF), lambda i: (i, 0)),
            full(W1), full(b1), full(W2), full(b2), full(W3q), full(b3q),
            full(S),
        ],
        out_specs=pl.BlockSpec((BE, DF), lambda i: (i, 0)),
        out_shape=jax.ShapeDtypeStruct((EP, DF), jnp.float32),
    )(eap, xs, W1, b1, W2, b2, W3q, b3q, S)


def _scatter_msg(msg, dstp, zeros128):
    @functools.partial(
        pl.kernel,
        out_type=jax.ShapeDtypeStruct((NC, NR, DF), jnp.float32),
        mesh=_sc_mesh(),
        scratch_types=[
            pltpu.VMEM((C,), jnp.int32),
            pltpu.VMEM((C, DF), jnp.float32),
            pltpu.VMEM((C,), jnp.int32),
            pltpu.VMEM((C, DF), jnp.float32),
            pltpu.SemaphoreType.DMA,
            pltpu.SemaphoreType.DMA,
            pltpu.VMEM_SHARED((NR, DF), jnp.float32),
        ],
    )
    def k(msg_hbm, dst_hbm, zero_hbm, out_hbm, idx_a, rows_a, idx_b, rows_b,
          gsem_a, gsem_b, acc):
        c = lax.axis_index("c")
        s = lax.axis_index("s")
        wid = c * NS + s
        pltpu.sync_copy(zero_hbm.at[pl.ds(s * RPT, RPT)],
                        acc.at[pl.ds(s * RPT, RPT)])
        plsc.subcore_barrier()

        def issue_g(ci, idx_v, rows_v, gsem):
            base = (wid * CPW + ci) * C
            pltpu.sync_copy(dst_hbm.at[pl.ds(base, C)], idx_v)
            pltpu.async_copy(msg_hbm.at[pl.ds(base, C)], rows_v, gsem)

        def drain_g(rows_v, gsem):
            pltpu.make_async_copy(msg_hbm.at[pl.ds(0, C)], rows_v, gsem).wait()

        issue_g(0, idx_a, rows_a, gsem_a)

        def pair(j, carry):
            issue_g(2 * j + 1, idx_b, rows_b, gsem_b)
            drain_g(rows_a, gsem_a)
            pltpu.sync_copy(rows_a, acc.at[idx_a], add=True)

            @pl.when(j < CPW // 2 - 1)
            def _():
                issue_g(2 * j + 2, idx_a, rows_a, gsem_a)

            drain_g(rows_b, gsem_b)
            pltpu.sync_copy(rows_b, acc.at[idx_b], add=True)
            return carry

        lax.fori_loop(0, CPW // 2, pair, 0)
        plsc.subcore_barrier()
        pltpu.sync_copy(acc.at[pl.ds(s * RPT, RPT)],
                        out_hbm.at[c, pl.ds(s * RPT, RPT)])

    return k(msg, dstp, zeros128)


def _gat_edges(xps, edt, sd4, zeros128):
    @functools.partial(
        pl.kernel,
        out_type=jax.ShapeDtypeStruct((NC, NR, DF), jnp.float32),
        mesh=_sc_mesh(),
        scratch_types=[
            pltpu.VMEM((2, C5), jnp.int32),
            pltpu.VMEM((2, C5), jnp.int32),
            pltpu.VMEM((C5, DF), jnp.float32),
            pltpu.VMEM((C5, DF), jnp.float32),
            pltpu.VMEM((C5, DF), jnp.float32),
            pltpu.VMEM((C5, DF), jnp.float32),
            pltpu.SemaphoreType.DMA,
            pltpu.SemaphoreType.DMA,
            pltpu.VMEM_SHARED((NR, DF), jnp.float32),
        ],
    )
    def k(xps_hbm, edt_hbm, sd_hbm, zero_hbm, out_hbm,
          sdidx_a, sdidx_b, rows_a, ed_a, rows_b, ed_b,
          sem_a, sem_b, acc):
        c = lax.axis_index("c")
        s = lax.axis_index("s")
        wid = c * NS + s
        pltpu.sync_copy(zero_hbm.at[pl.ds(s * RPT, RPT)],
                        acc.at[pl.ds(s * RPT, RPT)])
        plsc.subcore_barrier()

        def issue(ci, sdidx_v, rows_v, ed_v, sem):
            pltpu.sync_copy(sd_hbm.at[wid, ci], sdidx_v)
            pltpu.async_copy(xps_hbm.at[sdidx_v.at[0]], rows_v, sem)
            pltpu.async_copy(edt_hbm.at[sdidx_v.at[1]], ed_v, sem)

        def drain(rows_v, ed_v, sem):
            pltpu.make_async_copy(xps_hbm.at[pl.ds(0, C5)], rows_v, sem).wait()
            pltpu.make_async_copy(edt_hbm.at[pl.ds(0, C5)], ed_v, sem).wait()

        def process(sdidx_v, rows_v, ed_v):
            def edge(e, ec):
                t = rows_v[e, pl.ds(H1, 16)] + ed_v[e, pl.ds(0, 16)]
                al = jnp.exp(jnp.maximum(t, 0.2 * t))
                for c4 in range(H1 // 16):
                    sl = pl.ds(c4 * 16, 16)
                    rows_v[e, sl] = rows_v[e, sl] * al
                rows_v[e, pl.ds(H1, 16)] = al
                return ec

            lax.fori_loop(0, C5, edge, 0)
            pltpu.sync_copy(rows_v, acc.at[sdidx_v.at[1]], add=True)

        issue(0, sdidx_a, rows_a, ed_a, sem_a)

        def pair(j, carry):
            issue(2 * j + 1, sdidx_b, rows_b, ed_b, sem_b)
            drain(rows_a, ed_a, sem_a)
            process(sdidx_a, rows_a, ed_a)

            @pl.when(j < CPW5 // 2 - 1)
            def _():
                issue(2 * j + 2, sdidx_a, rows_a, ed_a, sem_a)

            drain(rows_b, ed_b, sem_b)
            process(sdidx_b, rows_b, ed_b)
            return carry

        lax.fori_loop(0, CPW5 // 2, pair, 0)
        plsc.subcore_barrier()
        pltpu.sync_copy(acc.at[pl.ds(s * RPT, RPT)],
                        out_hbm.at[c, pl.ds(s * RPT, RPT)])

    return k(xps, edt, sd4, zeros128)


def _bn(t, g, b):
    mu = jnp.mean(t, axis=0, keepdims=True)
    var = jnp.mean((t - mu) * (t - mu), axis=0, keepdims=True)
    return (t - mu) / jnp.sqrt(var + 1e-5) * g + b


def _gat_tables(h, gW, gas, gad):
    xp = _mmd(h, gW)
    es = _mmd(xp, gas)
    ed = _mmd(xp, gad)
    xps = jnp.concatenate([xp, jnp.broadcast_to(es, (N, 16)),
                           jnp.zeros((N, 48), jnp.float32)], axis=1)
    edt = jnp.concatenate([jnp.broadcast_to(ed, (N, 16)),
                           jnp.zeros((N, 112), jnp.float32)], axis=1)
    return xps, edt


def _node1(x, accp, root, nbias, bg, bb, gW, gas, gad):
    def body(x_ref, acc_ref, root_r, nb_r, bg_r, bb_r, gW_r, gas_r, gad_r,
             xps_ref, edt_ref):
        agg = acc_ref[0] + acc_ref[1]
        t = _mmd(x_ref[...], root_r[...]) + agg + nb_r[...]
        h1 = jnp.maximum(_bn(t, bg_r[...], bb_r[...]), 0.0)
        xps, edt = _gat_tables(h1, gW_r[...], gas_r[...], gad_r[...])
        xps_ref[...] = xps
        edt_ref[...] = edt

    return pl.pallas_call(
        body,
        out_shape=(jax.ShapeDtypeStruct((N, DF), jnp.float32),
                   jax.ShapeDtypeStruct((N, DF), jnp.float32)),
    )(x, accp, root, nbias, bg, bb, gW, gas, gad)


def _gat_combine(acc_ref, xps_ref, edt_ref, gb):
    num = acc_ref[0, :, :H1] + acc_ref[1, :, :H1]
    den = acc_ref[0, :, H1:H1 + 1] + acc_ref[1, :, H1:H1 + 1]
    xp = xps_ref[:, :H1]
    t = xps_ref[:, H1:H1 + 1] + edt_ref[:, 0:1]
    al = jnp.exp(jnp.maximum(t, 0.2 * t))
    num = num + al * xp
    den = den + al
    return num / (den + 1e-16) + gb


def _node2(acc, xps1, edt1, g1b, bg, bb, gW, gas, gad):
    def body(acc_ref, xps_ref, edt_ref, g1b_r, bg_r, bb_r, gW_r, gas_r,
             gad_r, xps2_ref, edt2_ref):
        o = _gat_combine(acc_ref, xps_ref, edt_ref, g1b_r[...])
        h2 = jnp.maximum(_bn(o, bg_r[...], bb_r[...]), 0.0)
        xps, edt = _gat_tables(h2, gW_r[...], gas_r[...], gad_r[...])
        xps2_ref[...] = xps
        edt2_ref[...] = edt

    return pl.pallas_call(
        body,
        out_shape=(jax.ShapeDtypeStruct((N, DF), jnp.float32),
                   jax.ShapeDtypeStruct((N, DF), jnp.float32)),
    )(acc, xps1, edt1, g1b, bg, bb, gW, gas, gad)


def _head(acc, xps2, edt2, batch2d, g2b, bg, bb, fc1W, fc1b, bn1g, bn1b,
          fc2W, fc2b, bn2g, bn2b, fc3W, fc3b, ngraphs):
    def body(acc_ref, xps_ref, edt_ref, bat_ref, g2b_r, bg_r, bb_r,
             fc1W_r, fc1b_r, bn1g_r, bn1b_r, fc2W_r, fc2b_r, bn2g_r,
             bn2b_r, fc3W_r, fc3b_r, out_ref):
        o = _gat_combine(acc_ref, xps_ref, edt_ref, g2b_r[...])
        h3 = jnp.maximum(_bn(o, bg_r[...], bb_r[...]), 0.0)
        gid = lax.broadcasted_iota(jnp.int32, (ngraphs, N), 0)
        oh = (bat_ref[...] == gid).astype(jnp.float32)
        sums = _mm(oh, h3)
        cnt = jnp.sum(oh, axis=1, keepdims=True)
        g = sums / jnp.maximum(cnt, 1.0)
        g = jnp.maximum(
            _bn(_mmd(g, fc1W_r[...]) + fc1b_r[...], bn1g_r[...], bn1b_r[...]), 0.0)
        g = jnp.maximum(
            _bn(_mmd(g, fc2W_r[...]) + fc2b_r[...], bn2g_r[...], bn2b_r[...]), 0.0)
        out_ref[...] = _mmd(g, fc3W_r[...]) + fc3b_r[...]

    return pl.pallas_call(
        body,
        out_shape=jax.ShapeDtypeStruct((ngraphs, 64), jnp.float32),
    )(acc, xps2, edt2, batch2d, g2b, bg, bb, fc1W, fc1b, bn1g, bn1b,
      fc2W, fc2b, bn2g, bn2b, fc3W, fc3b)


def kernel(x, edge_index, edge_attr, batch, params):
    p = params
    src = edge_index[0]
    dst = edge_index[1]
    pad = EP - E
    srcp = jnp.concatenate([src, jnp.zeros((pad,), jnp.int32)])
    dstp = jnp.concatenate([dst, jnp.full((pad,), N, jnp.int32)])
    sd4 = jnp.stack([srcp.reshape(NW, CPW5, C5),
                     dstp.reshape(NW, CPW5, C5)], axis=2)
    eap = jnp.concatenate([edge_attr, jnp.zeros((pad, DE), jnp.float32)])
    zeros128 = jnp.zeros((NR, DF), jnp.float32)
    batch2d = batch.reshape(1, N)

    r2 = lambda a: a.reshape(1, -1)
    W3q = p['emlp_W3'].reshape(64, DF, H0).transpose(0, 2, 1).reshape(64, DF * H0)
    b3q = p['emlp_b3'].reshape(DF, H0).T.reshape(1, DF * H0)

    S = (jnp.arange(DF * H0, dtype=jnp.int32)[:, None] // DF ==
         jnp.arange(H0, dtype=jnp.int32)[None, :]).astype(jnp.float32)
    xs = _gather_xs(x, srcp)
    msg = _edge_msg(eap, xs, p['emlp_W1'], r2(p['emlp_b1']),
                    p['emlp_W2'], r2(p['emlp_b2']), W3q, b3q, S)
    acc0 = _scatter_msg(msg, dstp, zeros128)
    xps1, edt1 = _node1(x, acc0[:, :N, :H0], p['nn_root'], r2(p['nn_bias']),
                        r2(p['bng1_g']), r2(p['bng1_b']),
                        p['gat1_W'], p['gat1_as'].reshape(-1, 1),
                        p['gat1_ad'].reshape(-1, 1))
    acc1 = _gat_edges(xps1, edt1, sd4, zeros128)
    xps2, edt2 = _node2(acc1[:, :N, :80], xps1, edt1[:, :8], r2(p['gat1_b']),
                        r2(p['bng2_g']), r2(p['bng2_b']),
                        p['gat2_W'], p['gat2_as'].reshape(-1, 1),
                        p['gat2_ad'].reshape(-1, 1))
    acc2 = _gat_edges(xps2, edt2, sd4, zeros128)
    out = _head(acc2[:, :N, :80], xps2, edt2[:, :8], batch2d, r2(p['gat2_b']),
                r2(p['bng3_g']), r2(p['bng3_b']),
                p['fc1_W'], r2(p['fc1_b']), r2(p['bn1_g']), r2(p['bn1_b']),
                p['fc2_W'], r2(p['fc2_b']), r2(p['bn2_g']), r2(p['bn2_b']),
                p['fc3_W'], r2(p['fc3_b']), 64)
    return out


# speedup vs baseline: 1.6966x; 1.0053x over previous
import functools

import jax
import jax.numpy as jnp
from jax import lax
from jax.experimental import pallas as pl
from jax.experimental.pallas import tpu as pltpu
from jax.experimental.pallas import tpu_sc as plsc

N = 10000
E = 160000
DF = 128
DE = 16
H0, H1, H2 = 16, 64, 64

NC, NS = 2, 16
NW = NC * NS
C = 128
CPW = 40
EP = NW * CPW * C
NR = 10240
RPT = NR // NS
C5 = 64
CPW5 = EP // (NW * C5)


def _mm(a, b):
    return jnp.matmul(a, b, precision=lax.Precision.HIGHEST)


def _mmd(a, b):
    return jnp.matmul(a.astype(jnp.bfloat16), b.astype(jnp.bfloat16),
                      preferred_element_type=jnp.float32)


def _bf(a):
    return a.astype(jnp.bfloat16).astype(jnp.float32)

def _sc_mesh():
    return plsc.VectorSubcoreMesh(core_axis_name="c", subcore_axis_name="s")


def _gather_xs(x, src3):
    @functools.partial(
        pl.kernel,
        out_type=jax.ShapeDtypeStruct((EP, DF), jnp.float32),
        mesh=_sc_mesh(),
        scratch_types=[
            pltpu.VMEM((CPW, 1, C), jnp.int32),
            pltpu.VMEM((C, DF), jnp.float32),
            pltpu.VMEM((C, DF), jnp.float32),
            pltpu.SemaphoreType.DMA,
            pltpu.SemaphoreType.DMA,
            pltpu.SemaphoreType.DMA,
            pltpu.SemaphoreType.DMA,
        ],
    )
    def k(x_hbm, src_hbm, out_hbm, idx_all, rows_a, rows_b,
          gsem_a, gsem_b, wsem_a, wsem_b):
        wid = lax.axis_index("c") * NS + lax.axis_index("s")
        pltpu.sync_copy(src_hbm.at[wid], idx_all)

        def issue_g(ci, rows_v, gsem):
            pltpu.async_copy(x_hbm.at[idx_all.at[ci, 0]], rows_v, gsem)

        def drain_g(rows_v, gsem):
            pltpu.make_async_copy(x_hbm.at[pl.ds(0, C)], rows_v, gsem).wait()

        def start_w(ci, rows_v, wsem):
            base = (wid * CPW + ci) * C
            pltpu.async_copy(rows_v, out_hbm.at[pl.ds(base, C)], wsem)

        def drain_w(rows_v, wsem):
            pltpu.make_async_copy(rows_v, out_hbm.at[pl.ds(0, C)], wsem).wait()

        issue_g(0, rows_a, gsem_a)

        def pair(j, carry):
            @pl.when(j > 0)
            def _():
                drain_w(rows_b, wsem_b)

            issue_g(2 * j + 1, rows_b, gsem_b)
            drain_g(rows_a, gsem_a)
            start_w(2 * j, rows_a, wsem_a)

            @pl.when(j < CPW // 2 - 1)
            def _():
                drain_w(rows_a, wsem_a)
                issue_g(2 * j + 2, rows_a, gsem_a)

            drain_g(rows_b, gsem_b)
            start_w(2 * j + 1, rows_b, wsem_b)
            return carry

        lax.fori_loop(0, CPW // 2, pair, 0)
        drain_w(rows_a, wsem_a)
        drain_w(rows_b, wsem_b)

    return k(x, src3)


def _edge_msg(eap, xs, W1, b1, W2, b2, W3q, b3q, S):
    BE = 512
    nblk = EP // BE

    def body(ea_ref, xs_ref, W1r, b1r, W2r, b2r, W3r, b3r, S_r, msg_ref):
        h = jnp.maximum(_mmd(ea_ref[...], W1r[...]) + b1r[...], 0.0)
        h = jnp.maximum(_mmd(h, W2r[...]) + b2r[...], 0.0)
        w = _mmd(h, W3r[...]) + b3r[...]
        xsb = _bf(xs_ref[...])
        wb = _bf(w)
        cols = [
            jnp.sum(wb[:, o * DF:(o + 1) * DF] * xsb, axis=1, keepdims=True)
            for o in range(H0)
        ]
        cols.append(jnp.zeros((BE, DF - H0), jnp.float32))
        msg_ref[...] = jnp.concatenate(cols, axis=1)

    full = lambda a: pl.BlockSpec(a.shape, lambda i: (0,) * a.ndim)
    return pl.pallas_call(
        body,
        grid=(nblk,),
        in_specs=[
            pl.BlockSpec((BE, DE), lambda i: (i, 0)),
            pl.BlockSpec((BE, DF), lambda i: (i, 0)),
            full(W1), full(b1), full(W2), full(b2), full(W3q), full(b3q),
            full(S),
        ],
        out_specs=pl.BlockSpec((BE, DF), lambda i: (i, 0)),
        out_shape=jax.ShapeDtypeStruct((EP, DF), jnp.float32),
    )(eap, xs, W1, b1, W2, b2, W3q, b3q, S)


def _scatter_msg(msg, dst3, zeros128):
    @functools.partial(
        pl.kernel,
        out_type=jax.ShapeDtypeStruct((NC, NR, DF), jnp.float32),
        mesh=_sc_mesh(),
        scratch_types=[
            pltpu.VMEM((CPW, 1, C), jnp.int32),
            pltpu.VMEM((C, DF), jnp.float32),
            pltpu.VMEM((C, DF), jnp.float32),
            pltpu.SemaphoreType.DMA,
            pltpu.SemaphoreType.DMA,
            pltpu.VMEM_SHARED((NR, DF), jnp.float32),
        ],
    )
    def k(msg_hbm, dst_hbm, zero_hbm, out_hbm, idx_all, rows_a, rows_b,
          gsem_a, gsem_b, acc):
        c = lax.axis_index("c")
        s = lax.axis_index("s")
        wid = c * NS + s
        pltpu.sync_copy(dst_hbm.at[wid], idx_all)
        pltpu.sync_copy(zero_hbm.at[pl.ds(s * RPT, RPT)],
                        acc.at[pl.ds(s * RPT, RPT)])
        plsc.subcore_barrier()

        def issue_g(ci, rows_v, gsem):
            base = (wid * CPW + ci) * C
            pltpu.async_copy(msg_hbm.at[pl.ds(base, C)], rows_v, gsem)

        def drain_g(rows_v, gsem):
            pltpu.make_async_copy(msg_hbm.at[pl.ds(0, C)], rows_v, gsem).wait()

        issue_g(0, rows_a, gsem_a)

        def pair(j, carry):
            issue_g(2 * j + 1, rows_b, gsem_b)
            drain_g(rows_a, gsem_a)
            pltpu.sync_copy(rows_a, acc.at[idx_all.at[2 * j, 0]], add=True)

            @pl.when(j < CPW // 2 - 1)
            def _():
                issue_g(2 * j + 2, rows_a, gsem_a)

            drain_g(rows_b, gsem_b)
            pltpu.sync_copy(rows_b, acc.at[idx_all.at[2 * j + 1, 0]], add=True)
            return carry

        lax.fori_loop(0, CPW // 2, pair, 0)
        plsc.subcore_barrier()
        pltpu.sync_copy(acc.at[pl.ds(s * RPT, RPT)],
                        out_hbm.at[c, pl.ds(s * RPT, RPT)])

    return k(msg, dst3, zeros128)


def _gat_edges(xps, edt, sd4, zeros128):
    @functools.partial(
        pl.kernel,
        out_type=jax.ShapeDtypeStruct((NC, NR, DF), jnp.float32),
        mesh=_sc_mesh(),
        scratch_types=[
            pltpu.VMEM((2, C5), jnp.int32),
            pltpu.VMEM((2, C5), jnp.int32),
            pltpu.VMEM((C5, DF), jnp.float32),
            pltpu.VMEM((C5, DF), jnp.float32),
            pltpu.VMEM((C5, DF), jnp.float32),
            pltpu.VMEM((C5, DF), jnp.float32),
            pltpu.SemaphoreType.DMA,
            pltpu.SemaphoreType.DMA,
            pltpu.VMEM_SHARED((NR, DF), jnp.float32),
        ],
    )
    def k(xps_hbm, edt_hbm, sd_hbm, zero_hbm, out_hbm,
          sdidx_a, sdidx_b, rows_a, ed_a, rows_b, ed_b,
          sem_a, sem_b, acc):
        c = lax.axis_index("c")
        s = lax.axis_index("s")
        wid = c * NS + s
        pltpu.sync_copy(zero_hbm.at[pl.ds(s * RPT, RPT)],
                        acc.at[pl.ds(s * RPT, RPT)])
        plsc.subcore_barrier()

        def issue(ci, sdidx_v, rows_v, ed_v, sem):
            pltpu.sync_copy(sd_hbm.at[wid, ci], sdidx_v)
            pltpu.async_copy(xps_hbm.at[sdidx_v.at[0]], rows_v, sem)
            pltpu.async_copy(edt_hbm.at[sdidx_v.at[1]], ed_v, sem)

        def drain(rows_v, ed_v, sem):
            pltpu.make_async_copy(xps_hbm.at[pl.ds(0, C5)], rows_v, sem).wait()
            pltpu.make_async_copy(edt_hbm.at[pl.ds(0, C5)], ed_v, sem).wait()

        def process(sdidx_v, rows_v, ed_v):
            def edge(e, ec):
                t = rows_v[e, pl.ds(H1, 16)] + ed_v[e, pl.ds(0, 16)]
                al = jnp.exp(jnp.maximum(t, 0.2 * t))
                for c4 in range(H1 // 16):
                    sl = pl.ds(c4 * 16, 16)
                    rows_v[e, sl] = rows_v[e, sl] * al
                rows_v[e, pl.ds(H1, 16)] = al
                return ec

            lax.fori_loop(0, C5, edge, 0)
            pltpu.sync_copy(rows_v, acc.at[sdidx_v.at[1]], add=True)

        issue(0, sdidx_a, rows_a, ed_a, sem_a)

        def pair(j, carry):
            issue(2 * j + 1, sdidx_b, rows_b, ed_b, sem_b)
            drain(rows_a, ed_a, sem_a)
            process(sdidx_a, rows_a, ed_a)

            @pl.when(j < CPW5 // 2 - 1)
            def _():
                issue(2 * j + 2, sdidx_a, rows_a, ed_a, sem_a)

            drain(rows_b, ed_b, sem_b)
            process(sdidx_b, rows_b, ed_b)
            return carry

        lax.fori_loop(0, CPW5 // 2, pair, 0)
        plsc.subcore_barrier()
        pltpu.sync_copy(acc.at[pl.ds(s * RPT, RPT)],
                        out_hbm.at[c, pl.ds(s * RPT, RPT)])

    return k(xps, edt, sd4, zeros128)


def _bn(t, g, b):
    mu = jnp.mean(t, axis=0, keepdims=True)
    var = jnp.mean((t - mu) * (t - mu), axis=0, keepdims=True)
    return (t - mu) / jnp.sqrt(var + 1e-5) * g + b


def _gat_tables(h, gW, gas, gad):
    xp = _mmd(h, gW)
    es = _mmd(xp, gas)
    ed = _mmd(xp, gad)
    xps = jnp.concatenate([xp, jnp.broadcast_to(es, (N, 16)),
                           jnp.zeros((N, 48), jnp.float32)], axis=1)
    edt = jnp.concatenate([jnp.broadcast_to(ed, (N, 16)),
                           jnp.zeros((N, 112), jnp.float32)], axis=1)
    return xps, edt


def _node1(x, accp, root, nbias, bg, bb, gW, gas, gad):
    def body(x_ref, acc_ref, root_r, nb_r, bg_r, bb_r, gW_r, gas_r, gad_r,
             xps_ref, edt_ref):
        agg = acc_ref[0] + acc_ref[1]
        t = _mmd(x_ref[...], root_r[...]) + agg + nb_r[...]
        h1 = jnp.maximum(_bn(t, bg_r[...], bb_r[...]), 0.0)
        xps, edt = _gat_tables(h1, gW_r[...], gas_r[...], gad_r[...])
        xps_ref[...] = xps
        edt_ref[...] = edt

    return pl.pallas_call(
        body,
        out_shape=(jax.ShapeDtypeStruct((N, DF), jnp.float32),
                   jax.ShapeDtypeStruct((N, DF), jnp.float32)),
    )(x, accp, root, nbias, bg, bb, gW, gas, gad)


def _gat_combine(acc_ref, xps_ref, edt_ref, gb):
    num = acc_ref[0, :, :H1] + acc_ref[1, :, :H1]
    den = acc_ref[0, :, H1:H1 + 1] + acc_ref[1, :, H1:H1 + 1]
    xp = xps_ref[:, :H1]
    t = xps_ref[:, H1:H1 + 1] + edt_ref[:, 0:1]
    al = jnp.exp(jnp.maximum(t, 0.2 * t))
    num = num + al * xp
    den = den + al
    return num / (den + 1e-16) + gb


def _node2(acc, xps1, edt1, g1b, bg, bb, gW, gas, gad):
    def body(acc_ref, xps_ref, edt_ref, g1b_r, bg_r, bb_r, gW_r, gas_r,
             gad_r, xps2_ref, edt2_ref):
        o = _gat_combine(acc_ref, xps_ref, edt_ref, g1b_r[...])
        h2 = jnp.maximum(_bn(o, bg_r[...], bb_r[...]), 0.0)
        xps, edt = _gat_tables(h2, gW_r[...], gas_r[...], gad_r[...])
        xps2_ref[...] = xps
        edt2_ref[...] = edt

    return pl.pallas_call(
        body,
        out_shape=(jax.ShapeDtypeStruct((N, DF), jnp.float32),
                   jax.ShapeDtypeStruct((N, DF), jnp.float32)),
    )(acc, xps1, edt1, g1b, bg, bb, gW, gas, gad)


def _head(acc, xps2, edt2, batch2d, g2b, bg, bb, fc1W, fc1b, bn1g, bn1b,
          fc2W, fc2b, bn2g, bn2b, fc3W, fc3b, ngraphs):
    def body(acc_ref, xps_ref, edt_ref, bat_ref, g2b_r, bg_r, bb_r,
             fc1W_r, fc1b_r, bn1g_r, bn1b_r, fc2W_r, fc2b_r, bn2g_r,
             bn2b_r, fc3W_r, fc3b_r, out_ref):
        o = _gat_combine(acc_ref, xps_ref, edt_ref, g2b_r[...])
        h3 = jnp.maximum(_bn(o, bg_r[...], bb_r[...]), 0.0)
        gid = lax.broadcasted_iota(jnp.int32, (ngraphs, N), 0)
        oh = (bat_ref[...] == gid).astype(jnp.float32)
        sums = _mm(oh, h3)
        cnt = jnp.sum(oh, axis=1, keepdims=True)
        g = sums / jnp.maximum(cnt, 1.0)
        g = jnp.maximum(
            _bn(_mmd(g, fc1W_r[...]) + fc1b_r[...], bn1g_r[...], bn1b_r[...]), 0.0)
        g = jnp.maximum(
            _bn(_mmd(g, fc2W_r[...]) + fc2b_r[...], bn2g_r[...], bn2b_r[...]), 0.0)
        out_ref[...] = _mmd(g, fc3W_r[...]) + fc3b_r[...]

    return pl.pallas_call(
        body,
        out_shape=jax.ShapeDtypeStruct((ngraphs, 64), jnp.float32),
    )(acc, xps2, edt2, batch2d, g2b, bg, bb, fc1W, fc1b, bn1g, bn1b,
      fc2W, fc2b, bn2g, bn2b, fc3W, fc3b)


def kernel(x, edge_index, edge_attr, batch, params):
    p = params
    src = edge_index[0]
    dst = edge_index[1]
    pad = EP - E
    srcp = jnp.concatenate([src, jnp.zeros((pad,), jnp.int32)])
    dstp = jnp.concatenate([dst, jnp.full((pad,), N, jnp.int32)])
    sd4 = jnp.stack([srcp.reshape(NW, CPW5, C5),
                     dstp.reshape(NW, CPW5, C5)], axis=2)
    src3 = srcp.reshape(NW, CPW, 1, C)
    dst3 = dstp.reshape(NW, CPW, 1, C)
    eap = jnp.concatenate([edge_attr, jnp.zeros((pad, DE), jnp.float32)])
    zeros128 = jnp.zeros((NR, DF), jnp.float32)
    batch2d = batch.reshape(1, N)

    r2 = lambda a: a.reshape(1, -1)
    W3q = p['emlp_W3'].reshape(64, DF, H0).transpose(0, 2, 1).reshape(64, DF * H0)
    b3q = p['emlp_b3'].reshape(DF, H0).T.reshape(1, DF * H0)

    S = (jnp.arange(DF * H0, dtype=jnp.int32)[:, None] // DF ==
         jnp.arange(H0, dtype=jnp.int32)[None, :]).astype(jnp.float32)
    xs = _gather_xs(x, src3)
    msg = _edge_msg(eap, xs, p['emlp_W1'], r2(p['emlp_b1']),
                    p['emlp_W2'], r2(p['emlp_b2']), W3q, b3q, S)
    acc0 = _scatter_msg(msg, dst3, zeros128)
    xps1, edt1 = _node1(x, acc0[:, :N, :H0], p['nn_root'], r2(p['nn_bias']),
                        r2(p['bng1_g']), r2(p['bng1_b']),
                        p['gat1_W'], p['gat1_as'].reshape(-1, 1),
                        p['gat1_ad'].reshape(-1, 1))
    acc1 = _gat_edges(xps1, edt1, sd4, zeros128)
    xps2, edt2 = _node2(acc1[:, :N, :80], xps1, edt1[:, :8], r2(p['gat1_b']),
                        r2(p['bng2_g']), r2(p['bng2_b']),
                        p['gat2_W'], p['gat2_as'].reshape(-1, 1),
                        p['gat2_ad'].reshape(-1, 1))
    acc2 = _gat_edges(xps2, edt2, sd4, zeros128)
    out = _head(acc2[:, :N, :80], xps2, edt2[:, :8], batch2d, r2(p['gat2_b']),
                r2(p['bng3_g']), r2(p['bng3_b']),
                p['fc1_W'], r2(p['fc1_b']), r2(p['bn1_g']), r2(p['bn1_b']),
                p['fc2_W'], r2(p['fc2_b']), r2(p['bn2_g']), r2(p['bn2_b']),
                p['fc3_W'], r2(p['fc3_b']), 64)
    return out
